# baseline (speedup 1.0000x reference)
import functools

import jax
import jax.numpy as jnp
from jax import lax
from jax.experimental import pallas as pl
from jax.experimental.pallas import tpu as pltpu
from jax.experimental.pallas import tpu_sc as plsc


def _mm_body(enc_ref, w_ref, b_ref, m_ref):
    m_ref[...] = lax.dot_general(
        enc_ref[...], w_ref[...],
        dimension_numbers=(((1,), (1,)), ((), ())),
        preferred_element_type=jnp.float32,
    ) + b_ref[...]


def _fused_table(enc_table, dec_w, dec_b2d):
    v_enc, _ = enc_table.shape
    v_dec, _ = dec_w.shape
    return pl.pallas_call(
        _mm_body,
        out_shape=jax.ShapeDtypeStruct((v_enc, v_dec), jnp.float32),
    )(enc_table, dec_w, dec_b2d)


@functools.lru_cache(maxsize=None)
def _make_gather(n_tokens: int, vocab: int, vocab_pad: int):
    info = plsc.get_sparse_core_info()
    nw = info.num_cores * info.num_subcores
    assert n_tokens % nw == 0
    b_per_w = n_tokens // nw
    chunk = 32
    assert b_per_w % chunk == 0
    n_chunks = b_per_w // chunk
    mesh = plsc.VectorSubcoreMesh(core_axis_name="c", subcore_axis_name="s")

    @functools.partial(
        pl.kernel,
        mesh=mesh,
        out_type=jax.ShapeDtypeStruct((n_tokens, vocab), jnp.float32),
        scratch_types=[
            pltpu.VMEM((b_per_w,), jnp.int32),
            pltpu.VMEM((chunk, vocab), jnp.float32),
            pltpu.SemaphoreType.DMA,
        ],
        compiler_params=pltpu.CompilerParams(use_tc_tiling_on_sc=False),
    )
    def gather_k(m_hbm, idx_hbm, out_hbm, idx_v, buf, sem):
        wid = lax.axis_index("s") * info.num_cores + lax.axis_index("c")
        base = wid * b_per_w
        pltpu.sync_copy(idx_hbm.at[pl.ds(base, b_per_w)], idx_v)

        def body(g, carry):
            i0 = g * chunk
            pltpu.async_copy(
                m_hbm.at[idx_v.at[pl.ds(i0, chunk)]], buf, sem).wait()
            pltpu.sync_copy(buf, out_hbm.at[pl.ds(base + i0, chunk)])
            return carry

        lax.fori_loop(0, n_chunks, body, 0)

    return gather_k


def kernel(_input, enc_table, dec_w, dec_b):
    b, l = _input.shape
    vocab = dec_w.shape[0]
    idx = _input.reshape(-1)
    m = _fused_table(enc_table, dec_w, dec_b.reshape(1, -1))
    out = _make_gather(b * l, vocab, vocab)(m, idx)
    return out.reshape(b, l, vocab)


# speedup vs baseline: 1.0332x; 1.0332x over previous
import functools

import jax
import jax.numpy as jnp
from jax import lax
from jax.experimental import pallas as pl
from jax.experimental.pallas import tpu as pltpu
from jax.experimental.pallas import tpu_sc as plsc


def _mm_body(enc_ref, w_ref, b_ref, m_ref):
    m_ref[...] = lax.dot_general(
        enc_ref[...], w_ref[...],
        dimension_numbers=(((1,), (1,)), ((), ())),
        preferred_element_type=jnp.float32,
    ) + b_ref[...]


def _fused_table(enc_table, dec_w, dec_b2d):
    v_enc, _ = enc_table.shape
    v_dec, _ = dec_w.shape
    return pl.pallas_call(
        _mm_body,
        out_shape=jax.ShapeDtypeStruct((v_enc, v_dec), jnp.float32),
    )(enc_table, dec_w, dec_b2d)


@functools.lru_cache(maxsize=None)
def _make_gather(n_tokens: int, vocab: int, vocab_pad: int):
    info = plsc.get_sparse_core_info()
    nw = info.num_cores * info.num_subcores
    assert n_tokens % nw == 0
    b_per_w = n_tokens // nw
    chunk = 40
    assert b_per_w % (2 * chunk) == 0
    n_pairs = b_per_w // (2 * chunk)
    mesh = plsc.VectorSubcoreMesh(core_axis_name="c", subcore_axis_name="s")

    @functools.partial(
        pl.kernel,
        mesh=mesh,
        out_type=jax.ShapeDtypeStruct((n_tokens, vocab), jnp.float32),
        scratch_types=[
            pltpu.VMEM((b_per_w,), jnp.int32),
            pltpu.VMEM((chunk, vocab), jnp.float32),
            pltpu.VMEM((chunk, vocab), jnp.float32),
            pltpu.SemaphoreType.DMA,
            pltpu.SemaphoreType.DMA,
            pltpu.SemaphoreType.DMA,
            pltpu.SemaphoreType.DMA,
        ],
        compiler_params=pltpu.CompilerParams(use_tc_tiling_on_sc=False),
    )
    def gather_k(m_hbm, idx_hbm, out_hbm, idx_v, buf0, buf1,
                 gs0, gs1, ws0, ws1):
        wid = lax.axis_index("s") * info.num_cores + lax.axis_index("c")
        base = wid * b_per_w
        bufs, gsems, wsems = (buf0, buf1), (gs0, gs1), (ws0, ws1)
        pltpu.sync_copy(idx_hbm.at[pl.ds(base, b_per_w)], idx_v)

        def start_gather(g, p):
            pltpu.make_async_copy(
                m_hbm.at[idx_v.at[pl.ds(g * chunk, chunk)]],
                bufs[p], gsems[p]).start()

        def wait_gather(p):
            pltpu.make_async_copy(
                m_hbm.at[pl.ds(0, chunk)], bufs[p], gsems[p]).wait()

        def start_wb(g, p):
            pltpu.make_async_copy(
                bufs[p], out_hbm.at[pl.ds(base + g * chunk, chunk)],
                wsems[p]).start()

        def wait_wb(p):
            pltpu.make_async_copy(
                bufs[p], out_hbm.at[pl.ds(base, chunk)], wsems[p]).wait()

        start_gather(0, 0)
        start_gather(1, 1)

        def body(t, carry):
            g0 = 2 * t
            for p in (0, 1):
                wait_gather(p)
                start_wb(g0 + p, p)
            for p in (0, 1):
                @pl.when(t < n_pairs - 1)
                def _():
                    wait_wb(p)
                    start_gather(g0 + 2 + p, p)
            return carry

        lax.fori_loop(0, n_pairs, body, 0)
        wait_wb(0)
        wait_wb(1)

    return gather_k


def kernel(_input, enc_table, dec_w, dec_b):
    b, l = _input.shape
    vocab = dec_w.shape[0]
    idx = _input.reshape(-1)
    m = _fused_table(enc_table, dec_w, dec_b.reshape(1, -1))
    out = _make_gather(b * l, vocab, vocab)(m, idx)
    return out.reshape(b, l, vocab)
